# baseline (speedup 1.0000x reference)
import functools
import math

import jax
import jax.numpy as jnp
from jax import lax
from jax.experimental import pallas as pl
from jax.experimental.pallas import tpu as pltpu
from jax.experimental.pallas import tpu_sc as plsc

_B, _R, _N, _D = 64, 100, 1000, 128
_H, _QD, _K = 8, 16, 16


_NC, _NS = 2, 16
_NW = _NC * _NS
_BPW = _B // _NW
_NSC = 16


def _sc_topk_store(dbuf_pg, kib_i, ql, qbase, bN):
    i32 = jnp.int32
    lane = lax.broadcasted_iota(i32, (16,), 0)
    big = jnp.full((16,), 1 << 20, i32)
    inf = jnp.float32(jnp.inf)

    def vmin16(v):
        for s in (8, 4, 2, 1):
            v = jnp.minimum(v, jnp.take(v, lane ^ s))
        return v

    def load64(base, off_hi):
        out = []
        for k in range(4):
            off = jnp.minimum(16 * k, off_hi)
            v = dbuf_pg[ql, pl.ds(base + off, 16)]
            out.append((v, base + off + lane))
        return out

    def pass1(c, cm):
        vs = load64(c * 64, 984 - 64 * c)
        m = vmin16(jnp.minimum(jnp.minimum(vs[0][0], vs[1][0]),
                               jnp.minimum(vs[2][0], vs[3][0])))
        return jnp.where(lane == c, m, cm)

    cmins = lax.fori_loop(0, _NSC, pass1, jnp.full((16,), inf))

    def round_body(r, carry):
        cm, kvec = carry
        m = vmin16(cm)
        cstar_v = vmin16(jnp.where(cm == m, lane, big))
        cstar = cstar_v[0]
        vs = load64(cstar * 64, 984 - 64 * cstar)
        cand = big
        for v, posk in vs:
            cand = jnp.minimum(cand, jnp.where(v == m, posk, big))
        pos_v = vmin16(cand)
        pos = pos_v[0]
        kvec = jnp.where(lane == r, pos_v, kvec)
        wb = jnp.minimum((pos >> 4) << 4, _N - 16)
        w = dbuf_pg[ql, pl.ds(wb, 16)]
        dbuf_pg[ql, pl.ds(wb, 16)] = jnp.where(wb + lane == pos_v, inf, w)
        newm = jnp.full((16,), inf)
        for v, posk in vs:
            newm = jnp.minimum(newm, jnp.where(posk == pos_v, inf, v))
        cm = jnp.where(lane == cstar_v, vmin16(newm), cm)
        return cm, kvec

    _, kvec = lax.fori_loop(0, _K, round_body, (cmins, jnp.zeros((16,), i32)))
    kib_i[pl.ds(16 * (qbase + ql), 16)] = kvec + bN
    return 0


def _make_sc_body(b0):
  def _sc_body(dist_hbm, enc_hbm, cur_hbm, knn_out, cur_out,
               cur_v, cidx, kib, dbuf, ebuf, cbuf,
               semd0, semd1, seme0, seme1, semo0, semo1, semcg, semco):
    wid = lax.axis_index("s") * _NC + lax.axis_index("c")
    bb = b0 + wid
    start = bb * _R
    base = pl.multiple_of((start // 8) * 8, 8)
    shift = start - base
    pltpu.sync_copy(cur_hbm.at[pl.ds(base, 104)], cur_v.at[pl.ds(0, 104)])
    semd = [semd0, semd1]
    seme = [seme0, seme1]
    semo = [semo0, semo1]
    ecp = [None, None]
    ocp = [None, None]
    ccp = []
    for i in range(1):
        b = bb
        bN = b * _N
        for g in range(7):
            qoff = 16 * g if g < 6 else 88
            cv = cur_v[pl.ds(shift + qoff, 16)]
            cidx[i, pl.ds(16 * g, 16)] = jnp.clip(cv, 0, _N - 1) + bN
        cg1 = pltpu.async_copy(enc_hbm.at[cidx.at[i, pl.ds(0, 96)]],
                               cbuf.at[i, pl.ds(0, 96)], semcg)
        cg2 = pltpu.async_copy(enc_hbm.at[cidx.at[i, pl.ds(96, 16)]],
                               cbuf.at[i, pl.ds(88, 16)], semcg)
        cp = pltpu.async_copy(dist_hbm.at[cidx.at[i, pl.ds(0, 16)]],
                              dbuf.at[0], semd[0])
        kib_i = kib.at[i]
        for g in range(7):
            cp.wait()
            if g < 6:
                cp = pltpu.async_copy(
                    dist_hbm.at[cidx.at[i, pl.ds(16 * (g + 1), 16)]],
                    dbuf.at[(g + 1) % 2], semd[(g + 1) % 2])
            lo, hi, qbase = (0, 16, 16 * g) if g < 6 else (8, 12, 88)
            dbuf_pg = dbuf.at[g % 2]
            lax.fori_loop(
                lo, hi, lambda ql, _, dp=dbuf_pg, ki=kib_i, qb=qbase:
                _sc_topk_store(dp, ki, ql, qb, bN), 0)
        cg1.wait()
        cg2.wait()
        ccp.append(pltpu.async_copy(cbuf.at[i, pl.ds(0, _R)],
                                    cur_out.at[pl.ds(wid * _R, _R)], semco))
        for j in range(13):
            n = 128 if j < 12 else _R * _K - 12 * 128
            p = j % 2
            if ocp[p] is not None:
                ocp[p].wait()
                ocp[p] = None
            ecp[p] = (pltpu.async_copy(
                enc_hbm.at[kib.at[i, pl.ds(128 * j, n)]],
                ebuf.at[p] if n == 128 else ebuf.at[p, pl.ds(0, n)],
                seme[p]), n, wid * _R * _K + 128 * j)
            q = (j + 1) % 2
            if ecp[q] is not None:
                gcp, gn, goff = ecp[q]
                gcp.wait()
                ecp[q] = None
                ocp[q] = pltpu.async_copy(
                    ebuf.at[q] if gn == 128 else ebuf.at[q, pl.ds(0, gn)],
                    knn_out.at[pl.ds(goff, gn)], semo[q])
        gcp, gn, goff = ecp[0]
        gcp.wait()
        ecp[0] = None
        ocp[0] = pltpu.async_copy(ebuf.at[0, pl.ds(0, gn)],
                                  knn_out.at[pl.ds(goff, gn)], semo[0])
    for d in ocp + ccp:
        if d is not None:
            d.wait()

  return _sc_body


def _sc_stage(dist2d, enc2d, cur_flat, b0):
    nb = _NW
    mesh = plsc.VectorSubcoreMesh(core_axis_name="c", subcore_axis_name="s")
    f = pl.kernel(
        _make_sc_body(b0),
        compiler_params=pltpu.CompilerParams(use_tc_tiling_on_sc=False),
        out_type=[
            jax.ShapeDtypeStruct((nb * _R * _K, _D), jnp.float32),
            jax.ShapeDtypeStruct((nb * _R, _D), jnp.float32),
        ],
        mesh=mesh,
        scratch_types=[
            pltpu.VMEM((112,), jnp.int32),
            pltpu.VMEM((1, 112), jnp.int32),
            pltpu.VMEM((1, 1664), jnp.int32),
            pltpu.VMEM((2, 16, _N), jnp.float32),
            pltpu.VMEM((2, 128, _D), jnp.float32),
            pltpu.VMEM((1, 104, _D), jnp.float32),
        ] + [pltpu.SemaphoreType.DMA] * 8,
    )
    knn2d, cur2d = f(dist2d, enc2d, cur_flat)
    return (knn2d.reshape(nb, _R, _K * _D), cur2d.reshape(nb, _R, _D))


def _dense_body(a_ref, cur_ref, enc_ref, mask_ref,
                wq_ref, wk_ref, wv_ref, wmh_ref, bmh_ref,
                w1_ref, b1_ref, w2_ref, b2_ref, out_ref):
    f32 = jnp.float32
    a = a_ref[0]
    cur = cur_ref[0]
    enc = enc_ref[0]
    mask = mask_ref[0]

    total = jnp.zeros((_R, _D), f32)
    cnt = jnp.zeros((_R,), f32)
    sks = []
    for k in range(_K):
        ak = a[:, k * _D:(k + 1) * _D]
        sk = jnp.sum(ak, axis=1)
        sks.append(sk)
        total = total + ak
        cnt = cnt + jnp.where(sk == 0.0, 0.0, 1.0)
    mean = total / jnp.clip(cnt, 1e-9, None)[:, None]

    w1 = w1_ref[...]
    h = jnp.broadcast_to(b1_ref[0], (_R, 5 * _D))
    for k in range(_K):
        ak = a[:, k * _D:(k + 1) * _D]
        bk = jnp.where((sks[k] == 0.0)[:, None], mean, ak)
        h = h + lax.dot_general(bk, w1[:, k * _D:(k + 1) * _D],
                                (((1,), (1,)), ((), ())),
                                preferred_element_type=f32)
    h = jnp.maximum(h, 0.0)
    unvis = lax.dot_general(h, w2_ref[...], (((1,), (1,)), ((), ())),
                            preferred_element_type=f32) + b2_ref[0]

    wq = wq_ref[...]
    q = (lax.dot_general(cur, wq[:, :_D], (((1,), (1,)), ((), ())),
                         preferred_element_type=f32)
         + lax.dot_general(unvis, wq[:, _D:], (((1,), (1,)), ((), ())),
                           preferred_element_type=f32))
    kk = lax.dot_general(enc, wk_ref[...], (((1,), (1,)), ((), ())),
                         preferred_element_type=f32)
    vv = lax.dot_general(enc, wv_ref[...], (((1,), (1,)), ((), ())),
                         preferred_element_type=f32)

    inv_sq = 1.0 / math.sqrt(float(_QD))
    outs = []
    for hh in range(_H):
        sl = slice(hh * _QD, (hh + 1) * _QD)
        s = lax.dot_general(q[:, sl], kk[:, sl], (((1,), (1,)), ((), ())),
                            preferred_element_type=f32) * inv_sq + mask
        s = s - jnp.max(s, axis=1, keepdims=True)
        e = jnp.exp(s)
        w = e / jnp.sum(e, axis=1, keepdims=True)
        outs.append(lax.dot_general(w, vv[:, sl], (((1,), (0,)), ((), ())),
                                    preferred_element_type=f32))
    att = jnp.concatenate(outs, axis=1)
    mh = lax.dot_general(att, wmh_ref[...], (((1,), (1,)), ((), ())),
                         preferred_element_type=f32) + bmh_ref[0]

    logits = lax.dot_general(mh, enc, (((1,), (1,)), ((), ())),
                             preferred_element_type=f32) / math.sqrt(float(_D))
    logits = 10.0 * jnp.tanh(logits) + mask
    logits = logits - jnp.max(logits, axis=1, keepdims=True)
    e = jnp.exp(logits)
    out_ref[0] = e / jnp.sum(e, axis=1, keepdims=True)


def _dense_stage(a_flat, cur_emb, encoded_nodes, ninf_mask,
                 Wq, Wk, Wv, Wmh, bmh, W1, b1, W2, b2, b0, nb):
    full = lambda shp: pl.BlockSpec(shp, lambda b: (0,) * len(shp))
    grid_spec = pl.GridSpec(
        grid=(nb,),
        in_specs=[
            pl.BlockSpec((1, _R, _K * _D), lambda b: (b, 0, 0)),
            pl.BlockSpec((1, _R, _D), lambda b: (b, 0, 0)),
            pl.BlockSpec((1, _N, _D), lambda b: (b0 + b, 0, 0)),
            pl.BlockSpec((1, _R, _N), lambda b: (b0 + b, 0, 0)),
            full((_H * _QD, 2 * _D)),
            full((_H * _QD, _D)),
            full((_H * _QD, _D)),
            full((_D, _H * _QD)),
            full((1, _D)),
            full((5 * _D, _K * _D)),
            full((1, 5 * _D)),
            full((_D, 5 * _D)),
            full((1, _D)),
        ],
        out_specs=pl.BlockSpec((1, _R, _N), lambda b: (b, 0, 0)),
    )
    return pl.pallas_call(
        _dense_body,
        grid_spec=grid_spec,
        out_shape=jax.ShapeDtypeStruct((nb, _R, _N), jnp.float32),
    )(a_flat, cur_emb, encoded_nodes, ninf_mask,
      Wq, Wk, Wv, Wmh, bmh.reshape(1, _D), W1, b1.reshape(1, 5 * _D),
      W2, b2.reshape(1, _D))


def kernel(encoded_nodes, distance, current, ninf_mask,
           Wq, Wk, Wv, Wmh, bmh, W1, b1, W2, b2):
    dist2d = distance.reshape(_B * _N, _N)
    enc2d = encoded_nodes.reshape(_B * _N, _D)
    cur_flat = current.reshape(-1)
    halves = []
    for b0 in (0, _NW):
        a_flat, cur_emb = _sc_stage(dist2d, enc2d, cur_flat, b0)
        halves.append(_dense_stage(
            a_flat, cur_emb, encoded_nodes, ninf_mask,
            Wq, Wk, Wv, Wmh, bmh, W1, b1, W2, b2, b0, _NW))
    return jnp.concatenate(halves, axis=0)


# speedup vs baseline: 1.0110x; 1.0110x over previous
import functools
import math

import jax
import jax.numpy as jnp
from jax import lax
from jax.experimental import pallas as pl
from jax.experimental.pallas import tpu as pltpu
from jax.experimental.pallas import tpu_sc as plsc

_B, _R, _N, _D = 64, 100, 1000, 128
_H, _QD, _K = 8, 16, 16


_NC, _NS = 2, 16
_NW = _NC * _NS
_BPW = _B // _NW
_NSC = 16


def _sc_topk_store(dbuf_pg, kib_i, ql, qbase, bN):
    i32 = jnp.int32
    lane = lax.broadcasted_iota(i32, (16,), 0)
    big = jnp.full((16,), 1 << 20, i32)
    inf = jnp.float32(jnp.inf)

    def vmin16(v):
        for s in (8, 4, 2, 1):
            v = jnp.minimum(v, jnp.take(v, lane ^ s))
        return v

    def load64(base, off_hi):
        out = []
        for k in range(4):
            off = jnp.minimum(16 * k, off_hi)
            v = dbuf_pg[ql, pl.ds(base + off, 16)]
            out.append((v, base + off + lane))
        return out

    def pass1(c, cm):
        vs = load64(c * 64, 984 - 64 * c)
        m = vmin16(jnp.minimum(jnp.minimum(vs[0][0], vs[1][0]),
                               jnp.minimum(vs[2][0], vs[3][0])))
        return jnp.where(lane == c, m, cm)

    cmins = lax.fori_loop(0, _NSC, pass1, jnp.full((16,), inf))

    def round_body(r, carry):
        cm, kvec = carry
        m = vmin16(cm)
        cstar_v = vmin16(jnp.where(cm == m, lane, big))
        cstar = cstar_v[0]
        vs = load64(cstar * 64, 984 - 64 * cstar)
        cand = big
        for v, posk in vs:
            cand = jnp.minimum(cand, jnp.where(v == m, posk, big))
        pos_v = vmin16(cand)
        pos = pos_v[0]
        kvec = jnp.where(lane == r, pos_v, kvec)
        wb = jnp.minimum((pos >> 4) << 4, _N - 16)
        w = dbuf_pg[ql, pl.ds(wb, 16)]
        dbuf_pg[ql, pl.ds(wb, 16)] = jnp.where(wb + lane == pos_v, inf, w)
        newm = jnp.full((16,), inf)
        for v, posk in vs:
            newm = jnp.minimum(newm, jnp.where(posk == pos_v, inf, v))
        cm = jnp.where(lane == cstar_v, vmin16(newm), cm)
        return cm, kvec

    _, kvec = lax.fori_loop(0, _K, round_body, (cmins, jnp.zeros((16,), i32)))
    kib_i[pl.ds(16 * (qbase + ql), 16)] = kvec + bN
    return 0


def _make_sc_body(b0):
  def _sc_body(dist_hbm, enc_hbm, cur_hbm, knn_out, cur_out,
               cur_v, cidx, kib, dbuf, ebuf, cbuf,
               semd0, semd1, seme0, seme1, semo0, semo1, semcg, semco):
    wid = lax.axis_index("s") * _NC + lax.axis_index("c")
    bb = b0 + _BPW * wid
    start = bb * _R
    base = pl.multiple_of((start // 8) * 8, 8)
    shift = start - base
    pltpu.sync_copy(cur_hbm.at[pl.ds(base, _BPW * _R + 8)],
                    cur_v.at[pl.ds(0, _BPW * _R + 8)])
    semd = [semd0, semd1]
    seme = [seme0, seme1]
    semo = [semo0, semo1]
    ecp = [None, None]
    ocp = [None, None]
    ccp = []
    for i in range(_BPW):
        b = bb + i
        bN = b * _N
        for g in range(7):
            qoff = 16 * g if g < 6 else 88
            cv = cur_v[pl.ds(shift + i * _R + qoff, 16)]
            cidx[i, pl.ds(16 * g, 16)] = jnp.clip(cv, 0, _N - 1) + bN
        cg1 = pltpu.async_copy(enc_hbm.at[cidx.at[i, pl.ds(0, 96)]],
                               cbuf.at[i, pl.ds(0, 96)], semcg)
        cg2 = pltpu.async_copy(enc_hbm.at[cidx.at[i, pl.ds(96, 16)]],
                               cbuf.at[i, pl.ds(88, 16)], semcg)
        cp = pltpu.async_copy(dist_hbm.at[cidx.at[i, pl.ds(0, 16)]],
                              dbuf.at[0], semd[0])
        kib_i = kib.at[i]
        for g in range(7):
            cp.wait()
            if g < 6:
                cp = pltpu.async_copy(
                    dist_hbm.at[cidx.at[i, pl.ds(16 * (g + 1), 16)]],
                    dbuf.at[(g + 1) % 2], semd[(g + 1) % 2])
            lo, hi, qbase = (0, 16, 16 * g) if g < 6 else (8, 12, 88)
            dbuf_pg = dbuf.at[g % 2]
            lax.fori_loop(
                lo, hi, lambda ql, _, dp=dbuf_pg, ki=kib_i, qb=qbase:
                _sc_topk_store(dp, ki, ql, qb, bN), 0)
        cg1.wait()
        cg2.wait()
        ccp.append(pltpu.async_copy(cbuf.at[i, pl.ds(0, _R)],
                                    cur_out.at[pl.ds((b - b0) * _R, _R)],
                                    semco))
        for j in range(13):
            n = 128 if j < 12 else _R * _K - 12 * 128
            p = j % 2
            if ocp[p] is not None:
                ocp[p].wait()
                ocp[p] = None
            ecp[p] = (pltpu.async_copy(
                enc_hbm.at[kib.at[i, pl.ds(128 * j, n)]],
                ebuf.at[p] if n == 128 else ebuf.at[p, pl.ds(0, n)],
                seme[p]), n, (b - b0) * _R * _K + 128 * j)
            q = (j + 1) % 2
            if ecp[q] is not None:
                gcp, gn, goff = ecp[q]
                gcp.wait()
                ecp[q] = None
                ocp[q] = pltpu.async_copy(
                    ebuf.at[q] if gn == 128 else ebuf.at[q, pl.ds(0, gn)],
                    knn_out.at[pl.ds(goff, gn)], semo[q])
        gcp, gn, goff = ecp[0]
        gcp.wait()
        ecp[0] = None
        ocp[0] = pltpu.async_copy(ebuf.at[0, pl.ds(0, gn)],
                                  knn_out.at[pl.ds(goff, gn)], semo[0])
    for d in ocp + ccp:
        if d is not None:
            d.wait()

  return _sc_body


def _sc_stage(dist2d, enc2d, cur_flat, b0):
    nb = _NW * _BPW
    mesh = plsc.VectorSubcoreMesh(core_axis_name="c", subcore_axis_name="s")
    f = pl.kernel(
        _make_sc_body(b0),
        compiler_params=pltpu.CompilerParams(use_tc_tiling_on_sc=False),
        out_type=[
            jax.ShapeDtypeStruct((nb * _R * _K, _D), jnp.float32),
            jax.ShapeDtypeStruct((nb * _R, _D), jnp.float32),
        ],
        mesh=mesh,
        scratch_types=[
            pltpu.VMEM((_BPW * _R + 8,), jnp.int32),
            pltpu.VMEM((_BPW, 112), jnp.int32),
            pltpu.VMEM((_BPW, 1664), jnp.int32),
            pltpu.VMEM((2, 16, _N), jnp.float32),
            pltpu.VMEM((2, 128, _D), jnp.float32),
            pltpu.VMEM((_BPW, 104, _D), jnp.float32),
        ] + [pltpu.SemaphoreType.DMA] * 8,
    )
    knn2d, cur2d = f(dist2d, enc2d, cur_flat)
    return (knn2d.reshape(nb, _R, _K * _D), cur2d.reshape(nb, _R, _D))


def _dense_body(a_ref, cur_ref, enc_ref, mask_ref,
                wq_ref, wk_ref, wv_ref, wmh_ref, bmh_ref,
                w1_ref, b1_ref, w2_ref, b2_ref, out_ref):
    f32 = jnp.float32
    a = a_ref[0]
    cur = cur_ref[0]
    enc = enc_ref[0]
    mask = mask_ref[0]

    total = jnp.zeros((_R, _D), f32)
    cnt = jnp.zeros((_R,), f32)
    sks = []
    for k in range(_K):
        ak = a[:, k * _D:(k + 1) * _D]
        sk = jnp.sum(ak, axis=1)
        sks.append(sk)
        total = total + ak
        cnt = cnt + jnp.where(sk == 0.0, 0.0, 1.0)
    mean = total / jnp.clip(cnt, 1e-9, None)[:, None]

    w1 = w1_ref[...]
    h = jnp.broadcast_to(b1_ref[0], (_R, 5 * _D))
    for k in range(_K):
        ak = a[:, k * _D:(k + 1) * _D]
        bk = jnp.where((sks[k] == 0.0)[:, None], mean, ak)
        h = h + lax.dot_general(bk, w1[:, k * _D:(k + 1) * _D],
                                (((1,), (1,)), ((), ())),
                                preferred_element_type=f32)
    h = jnp.maximum(h, 0.0)
    unvis = lax.dot_general(h, w2_ref[...], (((1,), (1,)), ((), ())),
                            preferred_element_type=f32) + b2_ref[0]

    wq = wq_ref[...]
    q = (lax.dot_general(cur, wq[:, :_D], (((1,), (1,)), ((), ())),
                         preferred_element_type=f32)
         + lax.dot_general(unvis, wq[:, _D:], (((1,), (1,)), ((), ())),
                           preferred_element_type=f32))
    kk = lax.dot_general(enc, wk_ref[...], (((1,), (1,)), ((), ())),
                         preferred_element_type=f32)
    vv = lax.dot_general(enc, wv_ref[...], (((1,), (1,)), ((), ())),
                         preferred_element_type=f32)

    inv_sq = 1.0 / math.sqrt(float(_QD))
    outs = []
    for hh in range(_H):
        sl = slice(hh * _QD, (hh + 1) * _QD)
        s = lax.dot_general(q[:, sl], kk[:, sl], (((1,), (1,)), ((), ())),
                            preferred_element_type=f32) * inv_sq + mask
        s = s - jnp.max(s, axis=1, keepdims=True)
        e = jnp.exp(s)
        w = e / jnp.sum(e, axis=1, keepdims=True)
        outs.append(lax.dot_general(w, vv[:, sl], (((1,), (0,)), ((), ())),
                                    preferred_element_type=f32))
    att = jnp.concatenate(outs, axis=1)
    mh = lax.dot_general(att, wmh_ref[...], (((1,), (1,)), ((), ())),
                         preferred_element_type=f32) + bmh_ref[0]

    logits = lax.dot_general(mh, enc, (((1,), (1,)), ((), ())),
                             preferred_element_type=f32) / math.sqrt(float(_D))
    logits = 10.0 * jnp.tanh(logits) + mask
    logits = logits - jnp.max(logits, axis=1, keepdims=True)
    e = jnp.exp(logits)
    out_ref[0] = e / jnp.sum(e, axis=1, keepdims=True)


def _dense_stage(a_flat, cur_emb, encoded_nodes, ninf_mask,
                 Wq, Wk, Wv, Wmh, bmh, W1, b1, W2, b2, b0, nb):
    full = lambda shp: pl.BlockSpec(shp, lambda b: (0,) * len(shp))
    grid_spec = pl.GridSpec(
        grid=(nb,),
        in_specs=[
            pl.BlockSpec((1, _R, _K * _D), lambda b: (b, 0, 0)),
            pl.BlockSpec((1, _R, _D), lambda b: (b, 0, 0)),
            pl.BlockSpec((1, _N, _D), lambda b: (b0 + b, 0, 0)),
            pl.BlockSpec((1, _R, _N), lambda b: (b0 + b, 0, 0)),
            full((_H * _QD, 2 * _D)),
            full((_H * _QD, _D)),
            full((_H * _QD, _D)),
            full((_D, _H * _QD)),
            full((1, _D)),
            full((5 * _D, _K * _D)),
            full((1, 5 * _D)),
            full((_D, 5 * _D)),
            full((1, _D)),
        ],
        out_specs=pl.BlockSpec((1, _R, _N), lambda b: (b, 0, 0)),
    )
    return pl.pallas_call(
        _dense_body,
        grid_spec=grid_spec,
        out_shape=jax.ShapeDtypeStruct((nb, _R, _N), jnp.float32),
    )(a_flat, cur_emb, encoded_nodes, ninf_mask,
      Wq, Wk, Wv, Wmh, bmh.reshape(1, _D), W1, b1.reshape(1, 5 * _D),
      W2, b2.reshape(1, _D))


def kernel(encoded_nodes, distance, current, ninf_mask,
           Wq, Wk, Wv, Wmh, bmh, W1, b1, W2, b2):
    dist2d = distance.reshape(_B * _N, _N)
    enc2d = encoded_nodes.reshape(_B * _N, _D)
    a_flat, cur_emb = _sc_stage(dist2d, enc2d, current.reshape(-1), 0)
    return _dense_stage(a_flat, cur_emb, encoded_nodes, ninf_mask,
                        Wq, Wk, Wv, Wmh, bmh, W1, b1, W2, b2, 0, _B)
